# Initial kernel scaffold; baseline (speedup 1.0000x reference)
import jax
import jax.numpy as jnp
from jax.experimental import pallas as pl


def kernel(h, codebook):
    raise NotImplementedError("write your pallas kernel here")


# speedup vs baseline: 1.1237x; 1.1237x over previous
import functools

import jax
import jax.numpy as jnp
from jax import lax
from jax.experimental import pallas as pl
from jax.experimental.pallas import tpu as pltpu
from jax.experimental.pallas import tpu_sc as plsc


def _argmin_body(bt, bc, n_emb, h_ref, cb_ref, idx_ref, dsum_ref):
    h_blk = h_ref[...]
    hs = jnp.sum(h_blk * h_blk, axis=1, keepdims=True)
    n_chunks = n_emb // bc

    def body(j, carry):
        bv, bi = carry
        cb = cb_ref[pl.ds(j * bc, bc), :]
        cs = jnp.sum(cb * cb, axis=1)
        s = lax.dot_general(h_blk, cb, (((1,), (1,)), ((), ())),
                            preferred_element_type=jnp.float32)
        d = (hs - 2.0 * s) + cs[None, :]
        m = jnp.min(d, axis=1, keepdims=True)
        ii = lax.broadcasted_iota(jnp.int32, (bt, bc), 1)
        li = jnp.min(jnp.where(d == m, ii, jnp.int32(2**30)),
                     axis=1, keepdims=True)
        upd = m < bv
        return (jnp.where(upd, m, bv),
                jnp.where(upd, j * bc + li, bi))

    bv0 = jnp.full((bt, 1), jnp.inf, dtype=jnp.float32)
    bi0 = jnp.zeros((bt, 1), dtype=jnp.int32)
    bv, bi = lax.fori_loop(0, n_chunks, body, (bv0, bi0))

    idx_ref[...] = bi.reshape(1, 1, bt)

    @pl.when(pl.program_id(0) == 0)
    def _():
        dsum_ref[0, 0] = 0.0
    dsum_ref[0, 0] += jnp.sum(bv)


def _make_argmin(n_tok, n_emb, d, bt, bc):
    grid = n_tok // bt
    return pl.pallas_call(
        functools.partial(_argmin_body, bt, bc, n_emb),
        grid=(grid,),
        in_specs=[
            pl.BlockSpec((bt, d), lambda i: (i, 0)),
            pl.BlockSpec((n_emb, d), lambda i: (0, 0)),
        ],
        out_specs=[
            pl.BlockSpec((1, 1, bt), lambda i: (i, 0, 0)),
            pl.BlockSpec(memory_space=pltpu.SMEM),
        ],
        out_shape=[
            jax.ShapeDtypeStruct((grid, 1, bt), jnp.int32),
            jax.ShapeDtypeStruct((1, 1), jnp.float32),
        ],
    )


_CHUNK = 128


def _make_gather(n_tok, n_emb, d):
    info = plsc.get_sparse_core_info()
    nw = info.num_cores * info.num_subcores
    bpw = n_tok // nw

    mesh = plsc.VectorSubcoreMesh(core_axis_name="c", subcore_axis_name="s")

    @functools.partial(
        pl.kernel, mesh=mesh,
        out_type=jax.ShapeDtypeStruct((n_tok, d), jnp.float32),
        scratch_types=[
            pltpu.VMEM((_CHUNK,), jnp.int32),
            pltpu.VMEM((_CHUNK, d), jnp.float32),
            pltpu.SemaphoreType.DMA,
        ],
    )
    def gather(table_hbm, idx_hbm, out_hbm, idx_v, rows_v, sem):
        wid = lax.axis_index("s") * info.num_cores + lax.axis_index("c")
        base = wid * bpw
        for j in range(bpw // _CHUNK):
            off = base + j * _CHUNK
            pltpu.sync_copy(idx_hbm.at[pl.ds(off, _CHUNK)], idx_v)
            pltpu.async_copy(table_hbm.at[idx_v], rows_v, sem).wait()
            pltpu.sync_copy(rows_v, out_hbm.at[pl.ds(off, _CHUNK)])

    return gather


def kernel(h, codebook):
    n_emb, d = codebook.shape
    h_flat = h.reshape(-1, d)
    n_tok = h_flat.shape[0]

    idx3, dsum = _make_argmin(n_tok, n_emb, d, bt=1024, bc=1024)(
        h_flat, codebook)
    indices = idx3.reshape(-1)

    quantized = _make_gather(n_tok, n_emb, d)(codebook, indices)
    quantized = quantized.reshape(h.shape)

    loss = dsum[0, 0] / jnp.float32(n_tok * d)
    return (quantized, 0.25 * loss, loss)


# speedup vs baseline: 1.6639x; 1.4807x over previous
import functools

import jax
import jax.numpy as jnp
from jax import lax
from jax.experimental import pallas as pl
from jax.experimental.pallas import tpu as pltpu
from jax.experimental.pallas import tpu_sc as plsc


def _argmin_body(bt, bc, n_emb, h_ref, cb_ref, idx_ref, dsum_ref,
                 cb2_ref, cs_ref):
    @pl.when(pl.program_id(0) == 0)
    def _():
        cb = cb_ref[...]
        cb2_ref[...] = cb + cb
        cs_ref[...] = jnp.sum(cb * cb, axis=1)[None, :]
        dsum_ref[0, 0] = 0.0

    h_blk = h_ref[...]
    hs = jnp.sum(h_blk * h_blk, axis=1, keepdims=True)
    hsb = jnp.broadcast_to(hs, (bt, 128))
    n_chunks = n_emb // bc
    nk = bc // 128

    def body(j, carry):
        val, kv = carry
        cb2 = cb2_ref[pl.ds(j * bc, bc), :]
        cs = cs_ref[:, pl.ds(j * bc, bc)]
        s2 = lax.dot_general(h_blk, cb2, (((1,), (1,)), ((), ())),
                             preferred_element_type=jnp.float32)
        for kk in range(nk):
            sl = slice(kk * 128, (kk + 1) * 128)
            dcol = (hsb - s2[:, sl]) + cs[:, sl]
            better = dcol < val
            val = jnp.where(better, dcol, val)
            kv = jnp.where(better, j * nk + kk, kv)
        return val, kv

    val0 = jnp.full((bt, 128), jnp.inf, dtype=jnp.float32)
    kv0 = jnp.zeros((bt, 128), dtype=jnp.int32)
    val, kv = lax.fori_loop(0, n_chunks, body, (val0, kv0))

    idx_full = kv * 128 + lax.broadcasted_iota(jnp.int32, (bt, 128), 1)
    m = jnp.min(val, axis=1, keepdims=True)
    li = jnp.min(jnp.where(val == m, idx_full, jnp.int32(2**30)),
                 axis=1, keepdims=True)
    idx_ref[...] = li.reshape(1, 1, bt)
    dsum_ref[0, 0] += jnp.sum(m)


def _make_argmin(n_tok, n_emb, d, bt, bc):
    grid = n_tok // bt
    return pl.pallas_call(
        functools.partial(_argmin_body, bt, bc, n_emb),
        grid=(grid,),
        in_specs=[
            pl.BlockSpec((bt, d), lambda i: (i, 0)),
            pl.BlockSpec((n_emb, d), lambda i: (0, 0)),
        ],
        out_specs=[
            pl.BlockSpec((1, 1, bt), lambda i: (i, 0, 0)),
            pl.BlockSpec(memory_space=pltpu.SMEM),
        ],
        out_shape=[
            jax.ShapeDtypeStruct((grid, 1, bt), jnp.int32),
            jax.ShapeDtypeStruct((1, 1), jnp.float32),
        ],
        scratch_shapes=[
            pltpu.VMEM((n_emb, d), jnp.float32),
            pltpu.VMEM((1, n_emb), jnp.float32),
        ],
    )


_CHUNK = 128


def _make_gather(n_tok, n_emb, d):
    info = plsc.get_sparse_core_info()
    nw = info.num_cores * info.num_subcores
    bpw = n_tok // nw

    mesh = plsc.VectorSubcoreMesh(core_axis_name="c", subcore_axis_name="s")

    @functools.partial(
        pl.kernel, mesh=mesh,
        out_type=jax.ShapeDtypeStruct((n_tok, d), jnp.float32),
        scratch_types=[
            pltpu.VMEM((_CHUNK,), jnp.int32),
            pltpu.VMEM((_CHUNK, d), jnp.float32),
            pltpu.SemaphoreType.DMA,
        ],
    )
    def gather(table_hbm, idx_hbm, out_hbm, idx_v, rows_v, sem):
        wid = lax.axis_index("s") * info.num_cores + lax.axis_index("c")
        base = wid * bpw
        for j in range(bpw // _CHUNK):
            off = base + j * _CHUNK
            pltpu.sync_copy(idx_hbm.at[pl.ds(off, _CHUNK)], idx_v)
            pltpu.async_copy(table_hbm.at[idx_v], rows_v, sem).wait()
            pltpu.sync_copy(rows_v, out_hbm.at[pl.ds(off, _CHUNK)])

    return gather


def kernel(h, codebook):
    n_emb, d = codebook.shape
    h_flat = h.reshape(-1, d)
    n_tok = h_flat.shape[0]

    idx3, dsum = _make_argmin(n_tok, n_emb, d, bt=2048, bc=2048)(
        h_flat, codebook)
    indices = idx3.reshape(-1)

    quantized = _make_gather(n_tok, n_emb, d)(codebook, indices)
    quantized = quantized.reshape(h.shape)

    loss = dsum[0, 0] / jnp.float32(n_tok * d)
    return (quantized, 0.25 * loss, loss)


# speedup vs baseline: 1.7001x; 1.0218x over previous
import functools

import jax
import jax.numpy as jnp
from jax import lax
from jax.experimental import pallas as pl
from jax.experimental.pallas import tpu as pltpu
from jax.experimental.pallas import tpu_sc as plsc


def _argmin_body(bt, bc, n_emb, h_ref, cb_ref, idx_ref, dsum_ref,
                 cb2_ref, cs_ref):
    @pl.when(pl.program_id(0) == 0)
    def _():
        cb = cb_ref[...]
        cb2_ref[...] = cb + cb
        cs_ref[...] = jnp.sum(cb * cb, axis=1)[None, :]
        dsum_ref[0, 0] = 0.0

    h_blk = h_ref[...]
    hs = jnp.sum(h_blk * h_blk, axis=1, keepdims=True)
    hsb = jnp.broadcast_to(hs, (bt, 128))
    n_chunks = n_emb // bc
    nk = bc // 128

    def body(j, carry):
        val, kv = carry
        cb2 = cb2_ref[pl.ds(j * bc, bc), :]
        cs = cs_ref[:, pl.ds(j * bc, bc)]
        s2 = lax.dot_general(h_blk, cb2, (((1,), (1,)), ((), ())),
                             preferred_element_type=jnp.float32)
        for kk in range(nk):
            sl = slice(kk * 128, (kk + 1) * 128)
            dcol = (hsb - s2[:, sl]) + cs[:, sl]
            better = dcol < val
            val = jnp.where(better, dcol, val)
            kv = jnp.where(better, j * nk + kk, kv)
        return val, kv

    val0 = jnp.full((bt, 128), jnp.inf, dtype=jnp.float32)
    kv0 = jnp.zeros((bt, 128), dtype=jnp.int32)
    val, kv = lax.fori_loop(0, n_chunks, body, (val0, kv0))

    idx_full = kv * 128 + lax.broadcasted_iota(jnp.int32, (bt, 128), 1)
    m = jnp.min(val, axis=1, keepdims=True)
    li = jnp.min(jnp.where(val == m, idx_full, jnp.int32(2**30)),
                 axis=1, keepdims=True)
    idx_ref[...] = li.reshape(1, 1, bt)
    dsum_ref[0, 0] += jnp.sum(m)


def _make_argmin(n_tok, n_emb, d, bt, bc):
    grid = n_tok // bt
    return pl.pallas_call(
        functools.partial(_argmin_body, bt, bc, n_emb),
        grid=(grid,),
        in_specs=[
            pl.BlockSpec((bt, d), lambda i: (i, 0)),
            pl.BlockSpec((n_emb, d), lambda i: (0, 0)),
        ],
        out_specs=[
            pl.BlockSpec((1, 1, bt), lambda i: (i, 0, 0)),
            pl.BlockSpec(memory_space=pltpu.SMEM),
        ],
        out_shape=[
            jax.ShapeDtypeStruct((grid, 1, bt), jnp.int32),
            jax.ShapeDtypeStruct((1, 1), jnp.float32),
        ],
        scratch_shapes=[
            pltpu.VMEM((n_emb, d), jnp.float32),
            pltpu.VMEM((1, n_emb), jnp.float32),
        ],
    )


_CHUNK = 128


def _make_gather(n_tok, n_emb, d):
    info = plsc.get_sparse_core_info()
    nw = info.num_cores * info.num_subcores
    bpw = n_tok // nw

    mesh = plsc.VectorSubcoreMesh(core_axis_name="c", subcore_axis_name="s")
    n_chunks = bpw // _CHUNK
    nb = 3

    @functools.partial(
        pl.kernel, mesh=mesh,
        out_type=jax.ShapeDtypeStruct((n_tok, d), jnp.float32),
        scratch_types=(
            [pltpu.VMEM((bpw,), jnp.int32)]
            + [pltpu.VMEM((_CHUNK, d), jnp.float32) for _ in range(nb)]
            + [pltpu.SemaphoreType.DMA for _ in range(2 * nb)]
        ),
    )
    def gather(table_hbm, idx_hbm, out_hbm, idx_v, *bufs_and_sems):
        bufs = bufs_and_sems[:nb]
        sem_g = bufs_and_sems[nb:2 * nb]
        sem_o = bufs_and_sems[2 * nb:]
        wid = lax.axis_index("s") * info.num_cores + lax.axis_index("c")
        base = wid * bpw
        pltpu.sync_copy(idx_hbm.at[pl.ds(base, bpw)], idx_v)
        g = [None] * n_chunks
        o = [None] * n_chunks
        for j in range(n_chunks):
            b = j % nb
            if j >= nb:
                o[j - nb].wait()
            g[j] = pltpu.async_copy(
                table_hbm.at[idx_v.at[pl.ds(j * _CHUNK, _CHUNK)]],
                bufs[b], sem_g[b])
            if j >= 1:
                g[j - 1].wait()
                o[j - 1] = pltpu.async_copy(
                    bufs[(j - 1) % nb],
                    out_hbm.at[pl.ds(base + (j - 1) * _CHUNK, _CHUNK)],
                    sem_o[(j - 1) % nb])
        j = n_chunks - 1
        g[j].wait()
        o[j] = pltpu.async_copy(
            bufs[j % nb], out_hbm.at[pl.ds(base + j * _CHUNK, _CHUNK)],
            sem_o[j % nb])
        for k in range(max(0, n_chunks - nb), n_chunks):
            o[k].wait()

    return gather


def kernel(h, codebook):
    n_emb, d = codebook.shape
    h_flat = h.reshape(-1, d)
    n_tok = h_flat.shape[0]

    idx3, dsum = _make_argmin(n_tok, n_emb, d, bt=2048, bc=2048)(
        h_flat, codebook)
    indices = idx3.reshape(-1)

    quantized = _make_gather(n_tok, n_emb, d)(codebook, indices)
    quantized = quantized.reshape(h.shape)

    loss = dsum[0, 0] / jnp.float32(n_tok * d)
    return (quantized, 0.25 * loss, loss)
